# baseline (speedup 1.0000x reference)
import functools

import jax
import jax.numpy as jnp
from jax import lax
from jax.experimental import pallas as pl
from jax.experimental.pallas import tpu as pltpu
from jax.experimental.pallas import tpu_sc as plsc

N = 10000
E = 320000
H = 128
G = 256
NZ = 100

NW = 32
CH = 256
NCH = 40
PW = NCH * CH
EP = NW * PW
NPAD = EP - E

BE = 2048
BN = 1000
RW = N // 16

_f32 = jnp.float32


def _silu(x):
    return x * jax.nn.sigmoid(x)


def _init_body(z_ref, emb_ref, w1a_ref, w1b_ref, b1_ref, h_ref, a_ref, b_ref):
    z = z_ref[...]
    oh = (lax.broadcasted_iota(jnp.int32, (BN, NZ), 1) == z).astype(_f32)
    h = jnp.dot(oh, emb_ref[...], preferred_element_type=_f32)
    h_ref[...] = h
    a_ref[...] = jnp.dot(h, w1a_ref[...],
                         preferred_element_type=_f32).astype(a_ref.dtype)
    b_ref[...] = (jnp.dot(h, w1b_ref[...], preferred_element_type=_f32)
                  + b1_ref[...]).astype(b_ref.dtype)


def _k_init(z2d, emb, w1a, w1b, b1):
    return pl.pallas_call(
        _init_body,
        grid=(N // BN,),
        in_specs=[
            pl.BlockSpec((BN, 1), lambda i: (i, 0)),
            pl.BlockSpec((NZ, H), lambda i: (0, 0)),
            pl.BlockSpec((H, H), lambda i: (0, 0)),
            pl.BlockSpec((H, H), lambda i: (0, 0)),
            pl.BlockSpec((1, H), lambda i: (0, 0)),
        ],
        out_specs=[pl.BlockSpec((BN, H), lambda i: (i, 0))] * 3,
        out_shape=[jax.ShapeDtypeStruct((N, H), _f32),
                   jax.ShapeDtypeStruct((N, H), jnp.bfloat16),
                   jax.ShapeDtypeStruct((N, H), jnp.bfloat16)],
    )(z2d, emb, w1a, w1b, b1)


_SC_MESH = plsc.VectorSubcoreMesh(core_axis_name="c", subcore_axis_name="s")

_bf16 = jnp.bfloat16
RPC = CH // 128


@functools.partial(
    pl.kernel,
    out_type=[
        jax.ShapeDtypeStruct((EP, H), _bf16),
        jax.ShapeDtypeStruct((EP, H), _bf16),
        jax.ShapeDtypeStruct((EP, 16), _f32),
        jax.ShapeDtypeStruct((EP, 16), _f32),
    ],
    mesh=_SC_MESH,
    scratch_types=[
        pltpu.VMEM((NCH * RPC, 128), jnp.int32),
        pltpu.VMEM((NCH * RPC, 128), jnp.int32),
        [pltpu.VMEM((CH, H), _bf16)] * 2,
        [pltpu.VMEM((CH, H), _bf16)] * 2,
        [pltpu.VMEM((CH, 16), _f32)] * 2,
        [pltpu.VMEM((CH, 16), _f32)] * 2,
        [pltpu.SemaphoreType.DMA] * 2,
        [pltpu.SemaphoreType.DMA] * 2,
    ],
    compiler_params=pltpu.CompilerParams(use_tc_tiling_on_sc=False),
)
def _gather_sc(a_hbm, b_hbm, p_hbm, src2_hbm, dst2_hbm,
               oa, ob, ops_, opd,
               sidx, didx, abuf, bbuf, psbuf, pdbuf, semg, semw):
    c = lax.axis_index("c")
    s = lax.axis_index("s")
    wid = s * 2 + c
    base = wid * PW
    rbase = wid * NCH * RPC
    pltpu.sync_copy(src2_hbm.at[pl.ds(rbase, NCH * RPC)], sidx)
    pltpu.sync_copy(dst2_hbm.at[pl.ds(rbase, NCH * RPC)], didx)

    def g_issue(j, b):
        for r in range(RPC):
            si = sidx.at[j * RPC + r]
            di = didx.at[j * RPC + r]
            sl = pl.ds(r * 128, 128)
            pltpu.async_copy(a_hbm.at[si], abuf[b].at[sl], semg[b])
            pltpu.async_copy(b_hbm.at[di], bbuf[b].at[sl], semg[b])
            pltpu.async_copy(p_hbm.at[si], psbuf[b].at[sl], semg[b])
            pltpu.async_copy(p_hbm.at[di], pdbuf[b].at[sl], semg[b])

    def g_drain(b):
        pltpu.make_async_copy(a_hbm.at[pl.ds(0, CH)], abuf[b], semg[b]).wait()
        pltpu.make_async_copy(b_hbm.at[pl.ds(0, CH)], bbuf[b], semg[b]).wait()
        pltpu.make_async_copy(p_hbm.at[pl.ds(0, CH)], psbuf[b], semg[b]).wait()
        pltpu.make_async_copy(p_hbm.at[pl.ds(0, CH)], pdbuf[b], semg[b]).wait()

    def w_issue(j, b):
        off = base + j * CH
        pltpu.async_copy(abuf[b], oa.at[pl.ds(off, CH)], semw[b])
        pltpu.async_copy(bbuf[b], ob.at[pl.ds(off, CH)], semw[b])
        pltpu.async_copy(psbuf[b], ops_.at[pl.ds(off, CH)], semw[b])
        pltpu.async_copy(pdbuf[b], opd.at[pl.ds(off, CH)], semw[b])

    def w_drain(b):
        pltpu.make_async_copy(oa.at[pl.ds(0, CH)], abuf[b], semw[b]).wait()
        pltpu.make_async_copy(ob.at[pl.ds(0, CH)], bbuf[b], semw[b]).wait()
        pltpu.make_async_copy(ops_.at[pl.ds(0, CH)], psbuf[b], semw[b]).wait()
        pltpu.make_async_copy(opd.at[pl.ds(0, CH)], pdbuf[b], semw[b]).wait()

    g_issue(0, 0)

    @pl.loop(0, NCH, step=2)
    def _pair(j0):
        for b in range(2):
            j = j0 + b
            g_drain(b)

            @pl.when(j >= 1)
            def _():
                w_drain(1 - b)

            @pl.when(j + 1 < NCH)
            def _():
                g_issue(j + 1, 1 - b)

            w_issue(j, b)

    w_drain(1)


def _edge_body(as_ref, bd_ref, ps_ref, pd_ref, w1c_ref, w2_ref, b2_ref,
               wc_ref, bc_ref, m2_ref, rw_ref):
    i = pl.program_id(0)
    rel = ps_ref[...] - pd_ref[...]
    d2 = jnp.sum(rel * rel, axis=1, keepdims=True)
    x = (as_ref[...].astype(_f32) + bd_ref[...].astype(_f32)
         + d2 * w1c_ref[...])
    m1 = _silu(x)
    m2 = _silu(jnp.dot(m1, w2_ref[...], preferred_element_type=_f32)
               + b2_ref[...])
    w = jnp.dot(m2, wc_ref[...], preferred_element_type=_f32) + bc_ref[...]
    relw = rel * w
    relw = jnp.where(lax.broadcasted_iota(jnp.int32, (BE, 16), 1) == 3,
                     1.0, relw)
    rid = i * BE + lax.broadcasted_iota(jnp.int32, (BE, 1), 0)
    valid = rid < E
    m2_ref[...] = jnp.where(valid, m2, 0.0).astype(m2_ref.dtype)
    rw_ref[...] = jnp.where(valid, relw, 0.0)


def _k_edge(a_s, b_d, p_s, p_d, w1c, w2, b2, wc, bc):
    return pl.pallas_call(
        _edge_body,
        grid=(EP // BE,),
        in_specs=[
            pl.BlockSpec((BE, H), lambda i: (i, 0)),
            pl.BlockSpec((BE, H), lambda i: (i, 0)),
            pl.BlockSpec((BE, 16), lambda i: (i, 0)),
            pl.BlockSpec((BE, 16), lambda i: (i, 0)),
            pl.BlockSpec((1, H), lambda i: (0, 0)),
            pl.BlockSpec((H, H), lambda i: (0, 0)),
            pl.BlockSpec((1, H), lambda i: (0, 0)),
            pl.BlockSpec((H, 1), lambda i: (0, 0)),
            pl.BlockSpec((1, 1), lambda i: (0, 0)),
        ],
        out_specs=[
            pl.BlockSpec((BE, H), lambda i: (i, 0)),
            pl.BlockSpec((BE, 16), lambda i: (i, 0)),
        ],
        out_shape=[
            jax.ShapeDtypeStruct((EP, H), jnp.bfloat16),
            jax.ShapeDtypeStruct((EP, 16), _f32),
        ],
    )(a_s, b_d, p_s, p_d, w1c, w2, b2, wc, bc)


SCH = 128
NSCH = PW // SCH


@functools.partial(
    pl.kernel,
    out_type=[
        jax.ShapeDtypeStruct((2, N, H), _bf16),
        jax.ShapeDtypeStruct((2, N, 16), _f32),
    ],
    mesh=_SC_MESH,
    scratch_types=[
        pltpu.VMEM((NSCH, 128), jnp.int32),
        [pltpu.VMEM((SCH, H), _bf16)] * 2,
        [pltpu.VMEM((SCH, 16), _f32)] * 2,
        pltpu.VMEM_SHARED((N, H), _bf16),
        pltpu.VMEM_SHARED((N, 16), _f32),
        [pltpu.SemaphoreType.DMA] * 2,
    ],
    compiler_params=pltpu.CompilerParams(use_tc_tiling_on_sc=False),
)
def _scatter_sc(m2_hbm, rw_hbm, dst2_hbm,
                om, oc,
                didx, m2buf, rwbuf, accm, accc, seml):
    c = lax.axis_index("c")
    s = lax.axis_index("s")
    wid = s * 2 + c
    base = wid * PW
    rbase = wid * NSCH
    rows0 = s * RW

    def l_issue(j, b):
        off = base + j * SCH
        pltpu.async_copy(m2_hbm.at[pl.ds(off, SCH)], m2buf[b], seml[b])
        pltpu.async_copy(rw_hbm.at[pl.ds(off, SCH)], rwbuf[b], seml[b])

    def l_drain(b):
        pltpu.make_async_copy(m2_hbm.at[pl.ds(0, SCH)], m2buf[b],
                              seml[b]).wait()
        pltpu.make_async_copy(rw_hbm.at[pl.ds(0, SCH)], rwbuf[b],
                              seml[b]).wait()

    l_issue(0, 0)
    pltpu.sync_copy(dst2_hbm.at[pl.ds(rbase, NSCH)], didx)

    @pl.loop(0, SCH)
    def _zrow(i):
        for g in range(H // 32):
            m2buf[1][i, pl.ds(g * 32, 32)] = jnp.zeros((32,), _bf16)
        rwbuf[1][i, pl.ds(0, 16)] = jnp.zeros((16,), _f32)

    for off, nrow in ((0, SCH), (SCH, SCH), (2 * SCH, SCH), (3 * SCH, SCH),
                      (4 * SCH, RW - 4 * SCH)):
        pltpu.sync_copy(m2buf[1].at[pl.ds(0, nrow)],
                        accm.at[pl.ds(rows0 + off, nrow)])
        pltpu.sync_copy(rwbuf[1].at[pl.ds(0, nrow)],
                        accc.at[pl.ds(rows0 + off, nrow)])
    plsc.subcore_barrier()

    @pl.loop(0, NSCH, step=2)
    def _pair(j0):
        for b in range(2):
            j = j0 + b
            l_drain(b)

            @pl.when(j + 1 < NSCH)
            def _():
                l_issue(j + 1, 1 - b)

            di = didx.at[j]
            pltpu.sync_copy(m2buf[b], accm.at[di], add=True)
            pltpu.sync_copy(rwbuf[b], accc.at[di], add=True)

    plsc.subcore_barrier()
    pltpu.sync_copy(accm.at[pl.ds(rows0, RW)], om.at[c, pl.ds(rows0, RW)])
    pltpu.sync_copy(accc.at[pl.ds(rows0, RW)], oc.at[c, pl.ds(rows0, RW)])


def _node_body(h_ref, aggp_ref, coordp_ref, pos_ref,
               wn1a_ref, wn1b_ref, bn1_ref, wn2_ref, bn2_ref,
               w1a_ref, w1b_ref, b1_ref,
               h_out, pos_out, a_out, b_out, *, last):
    agg = aggp_ref[0].astype(_f32) + aggp_ref[1].astype(_f32)
    hid = _silu(jnp.dot(h_ref[...], wn1a_ref[...], preferred_element_type=_f32)
                + jnp.dot(agg, wn1b_ref[...], preferred_element_type=_f32)
                + bn1_ref[...])
    hn = h_ref[...] + jnp.dot(hid, wn2_ref[...],
                              preferred_element_type=_f32) + bn2_ref[...]
    h_out[...] = hn
    if not last:
        coord = coordp_ref[0] + coordp_ref[1]
        deg = coord[:, 3:4] + 1.0
        posn = pos_ref[...] + coord / deg
        posn = jnp.where(
            lax.broadcasted_iota(jnp.int32, (BN, 16), 1) >= 3, 0.0, posn)
        pos_out[...] = posn
        a_out[...] = jnp.dot(hn, w1a_ref[...],
                             preferred_element_type=_f32).astype(a_out.dtype)
        b_out[...] = (jnp.dot(hn, w1b_ref[...], preferred_element_type=_f32)
                      + b1_ref[...]).astype(b_out.dtype)


def _k_node(h, aggp, coordp, pos, wn1a, wn1b, bn1, wn2, bn2, w1a, w1b, b1,
            last):
    n_out = 1 if last else 4
    out_specs = [pl.BlockSpec((BN, H), lambda i: (i, 0)),
                 pl.BlockSpec((BN, 16), lambda i: (i, 0)),
                 pl.BlockSpec((BN, H), lambda i: (i, 0)),
                 pl.BlockSpec((BN, H), lambda i: (i, 0))][:n_out]
    out_shape = [jax.ShapeDtypeStruct((N, H), _f32),
                 jax.ShapeDtypeStruct((N, 16), _f32),
                 jax.ShapeDtypeStruct((N, H), jnp.bfloat16),
                 jax.ShapeDtypeStruct((N, H), jnp.bfloat16)][:n_out]

    def body(*refs):
        ins = refs[:12]
        outs = list(refs[12:]) + [None] * (4 - n_out)
        _node_body(*ins, *outs, last=last)

    return pl.pallas_call(
        body,
        grid=(N // BN,),
        in_specs=[
            pl.BlockSpec((BN, H), lambda i: (i, 0)),
            pl.BlockSpec((2, BN, H), lambda i: (0, i, 0)),
            pl.BlockSpec((2, BN, 16), lambda i: (0, i, 0)),
            pl.BlockSpec((BN, 16), lambda i: (i, 0)),
            pl.BlockSpec((H, H), lambda i: (0, 0)),
            pl.BlockSpec((H, H), lambda i: (0, 0)),
            pl.BlockSpec((1, H), lambda i: (0, 0)),
            pl.BlockSpec((H, H), lambda i: (0, 0)),
            pl.BlockSpec((1, H), lambda i: (0, 0)),
            pl.BlockSpec((H, H), lambda i: (0, 0)),
            pl.BlockSpec((H, H), lambda i: (0, 0)),
            pl.BlockSpec((1, H), lambda i: (0, 0)),
        ],
        out_specs=out_specs,
        out_shape=out_shape,
    )(h, aggp, coordp, pos, wn1a, wn1b, bn1, wn2, bn2, w1a, w1b, b1)


def _pool_body(bt_ref, h_ref, o1_ref, bo1_ref, o2_ref, bo2_ref, out_ref,
               acc_ref):
    i = pl.program_id(0)

    @pl.when(i == 0)
    def _():
        acc_ref[...] = jnp.zeros((G, H), _f32)

    bt = bt_ref[...]
    oh = (lax.broadcasted_iota(jnp.int32, (BN, G), 1) == bt).astype(_f32)
    acc_ref[...] += lax.dot_general(oh, h_ref[...], (((0,), (0,)), ((), ())),
                                    preferred_element_type=_f32)

    @pl.when(i == pl.num_programs(0) - 1)
    def _():
        hg = acc_ref[...]
        out_ref[...] = jnp.dot(
            _silu(jnp.dot(hg, o1_ref[...], preferred_element_type=_f32)
                  + bo1_ref[...]),
            o2_ref[...], preferred_element_type=_f32) + bo2_ref[...]


def _k_pool(bt2d, h, o1, bo1, o2, bo2):
    return pl.pallas_call(
        _pool_body,
        grid=(N // BN,),
        in_specs=[
            pl.BlockSpec((BN, 1), lambda i: (i, 0)),
            pl.BlockSpec((BN, H), lambda i: (i, 0)),
            pl.BlockSpec((H, H), lambda i: (0, 0)),
            pl.BlockSpec((1, H), lambda i: (0, 0)),
            pl.BlockSpec((H, 1), lambda i: (0, 0)),
            pl.BlockSpec((1, 1), lambda i: (0, 0)),
        ],
        out_specs=pl.BlockSpec((G, 1), lambda i: (0, 0)),
        out_shape=jax.ShapeDtypeStruct((G, 1), _f32),
        scratch_shapes=[pltpu.VMEM((G, H), _f32)],
    )(bt2d, h, o1, bo1, o2, bo2)


def kernel(z, pos, edge_index, batch, params):
    conf_idx = 2
    pos3 = pos[:, conf_idx, :]
    posp = jnp.zeros((N, 16), _f32).at[:, :3].set(pos3)

    pad_idx = (jnp.arange(NPAD, dtype=jnp.int32) % N)
    srcp = jnp.concatenate([edge_index[0].astype(jnp.int32),
                            pad_idx]).reshape(EP // 128, 128)
    dstp = jnp.concatenate([edge_index[1].astype(jnp.int32),
                            pad_idx]).reshape(EP // 128, 128)

    z2d = z.astype(jnp.int32).reshape(N, 1)
    bt2d = batch.astype(jnp.int32).reshape(N, 1)
    z128 = jnp.zeros((N, H), _f32)
    z16 = jnp.zeros((N, 16), _f32)

    layers = params["layers"]

    def edge1_parts(layer):
        w1 = layer["edge1"]["W"]
        return (w1[:H], w1[H:2 * H], layer["edge1"]["b"].reshape(1, H),
                w1[2 * H].reshape(1, H))

    w1a0, w1b0, b10, _ = edge1_parts(layers[0])
    h, a, b = _k_init(z2d, params["embed"], w1a0, w1b0, b10)

    for li, layer in enumerate(layers):
        _, _, _, w1c = edge1_parts(layer)
        a_s, b_d, p_s, p_d = _gather_sc(a, b, posp, srcp, dstp)
        m2, relw = _k_edge(
            a_s, b_d, p_s, p_d, w1c,
            layer["edge2"]["W"], layer["edge2"]["b"].reshape(1, H),
            layer["coord"]["W"], layer["coord"]["b"].reshape(1, 1))
        aggp, coordp = _scatter_sc(m2, relw, dstp)

        last = li == len(layers) - 1
        wn1 = layer["node1"]["W"]
        if last:
            nw1a = nw1b = wn1[:H]
            nb1 = layer["node1"]["b"].reshape(1, H)
            outs = _k_node(h, aggp, coordp, posp,
                           wn1[:H], wn1[H:], nb1,
                           layer["node2"]["W"],
                           layer["node2"]["b"].reshape(1, H),
                           nw1a, nw1b, nb1, last=True)
            h = outs[0]
        else:
            w1an, w1bn, b1n, _ = edge1_parts(layers[li + 1])
            h, posp, a, b = _k_node(h, aggp, coordp, posp,
                                    wn1[:H], wn1[H:],
                                    layer["node1"]["b"].reshape(1, H),
                                    layer["node2"]["W"],
                                    layer["node2"]["b"].reshape(1, H),
                                    w1an, w1bn, b1n, last=False)

    out = _k_pool(bt2d, h, params["out1"]["W"],
                  params["out1"]["b"].reshape(1, H),
                  params["out2"]["W"], params["out2"]["b"].reshape(1, 1))
    return out


# speedup vs baseline: 1.7166x; 1.7166x over previous
import functools

import jax
import jax.numpy as jnp
from jax import lax
from jax.experimental import pallas as pl
from jax.experimental.pallas import tpu as pltpu
from jax.experimental.pallas import tpu_sc as plsc

N = 10000
E = 320000
H = 128
G = 256
NZ = 100

NW = 32
CH = 128
NCH = 80
PW = NCH * CH
EP = NW * PW
NPAD = EP - E

BE = 2048
BN = 1000
RW = N // 16

_f32 = jnp.float32


def _silu(x):
    return x * jax.nn.sigmoid(x)


def _init_body(z_ref, emb_ref, w1a_ref, w1b_ref, b1_ref, h_ref, a_ref, b_ref):
    z = z_ref[...]
    oh = (lax.broadcasted_iota(jnp.int32, (BN, NZ), 1) == z).astype(_f32)
    h = jnp.dot(oh, emb_ref[...], preferred_element_type=_f32)
    h_ref[...] = h
    a_ref[...] = jnp.dot(h, w1a_ref[...],
                         preferred_element_type=_f32).astype(a_ref.dtype)
    b_ref[...] = (jnp.dot(h, w1b_ref[...], preferred_element_type=_f32)
                  + b1_ref[...]).astype(b_ref.dtype)


def _k_init(z2d, emb, w1a, w1b, b1):
    return pl.pallas_call(
        _init_body,
        grid=(N // BN,),
        in_specs=[
            pl.BlockSpec((BN, 1), lambda i: (i, 0)),
            pl.BlockSpec((NZ, H), lambda i: (0, 0)),
            pl.BlockSpec((H, H), lambda i: (0, 0)),
            pl.BlockSpec((H, H), lambda i: (0, 0)),
            pl.BlockSpec((1, H), lambda i: (0, 0)),
        ],
        out_specs=[pl.BlockSpec((BN, H), lambda i: (i, 0))] * 3,
        out_shape=[jax.ShapeDtypeStruct((N, H), _f32)] * 3,
    )(z2d, emb, w1a, w1b, b1)


_SC_MESH = plsc.VectorSubcoreMesh(core_axis_name="c", subcore_axis_name="s")

_bf16 = jnp.bfloat16
RPC = CH // 128


@functools.partial(
    pl.kernel,
    out_type=[
        jax.ShapeDtypeStruct((EP, H), _f32),
        jax.ShapeDtypeStruct((EP, H), _f32),
        jax.ShapeDtypeStruct((EP, 16), _f32),
        jax.ShapeDtypeStruct((EP, 16), _f32),
    ],
    mesh=_SC_MESH,
    scratch_types=[
        pltpu.VMEM((NCH * RPC, 128), jnp.int32),
        pltpu.VMEM((NCH * RPC, 128), jnp.int32),
        [pltpu.VMEM((CH, H), _f32)] * 2,
        [pltpu.VMEM((CH, H), _f32)] * 2,
        [pltpu.VMEM((CH, 16), _f32)] * 2,
        [pltpu.VMEM((CH, 16), _f32)] * 2,
        [pltpu.SemaphoreType.DMA] * 2,
        [pltpu.SemaphoreType.DMA] * 2,
    ],
    compiler_params=pltpu.CompilerParams(use_tc_tiling_on_sc=False),
)
def _gather_sc(a_hbm, b_hbm, p_hbm, src2_hbm, dst2_hbm,
               oa, ob, ops_, opd,
               sidx, didx, abuf, bbuf, psbuf, pdbuf, semg, semw):
    c = lax.axis_index("c")
    s = lax.axis_index("s")
    wid = s * 2 + c
    base = wid * PW
    rbase = wid * NCH * RPC
    pltpu.sync_copy(src2_hbm.at[pl.ds(rbase, NCH * RPC)], sidx)
    pltpu.sync_copy(dst2_hbm.at[pl.ds(rbase, NCH * RPC)], didx)

    def g_issue(j, b):
        for r in range(RPC):
            si = sidx.at[j * RPC + r]
            di = didx.at[j * RPC + r]
            sl = pl.ds(r * 128, 128)
            pltpu.async_copy(a_hbm.at[si], abuf[b].at[sl], semg[b])
            pltpu.async_copy(b_hbm.at[di], bbuf[b].at[sl], semg[b])
            pltpu.async_copy(p_hbm.at[si], psbuf[b].at[sl], semg[b])
            pltpu.async_copy(p_hbm.at[di], pdbuf[b].at[sl], semg[b])

    def g_drain(b):
        pltpu.make_async_copy(a_hbm.at[pl.ds(0, CH)], abuf[b], semg[b]).wait()
        pltpu.make_async_copy(b_hbm.at[pl.ds(0, CH)], bbuf[b], semg[b]).wait()
        pltpu.make_async_copy(p_hbm.at[pl.ds(0, CH)], psbuf[b], semg[b]).wait()
        pltpu.make_async_copy(p_hbm.at[pl.ds(0, CH)], pdbuf[b], semg[b]).wait()

    def w_issue(j, b):
        off = base + j * CH
        pltpu.async_copy(abuf[b], oa.at[pl.ds(off, CH)], semw[b])
        pltpu.async_copy(bbuf[b], ob.at[pl.ds(off, CH)], semw[b])
        pltpu.async_copy(psbuf[b], ops_.at[pl.ds(off, CH)], semw[b])
        pltpu.async_copy(pdbuf[b], opd.at[pl.ds(off, CH)], semw[b])

    def w_drain(b):
        pltpu.make_async_copy(oa.at[pl.ds(0, CH)], abuf[b], semw[b]).wait()
        pltpu.make_async_copy(ob.at[pl.ds(0, CH)], bbuf[b], semw[b]).wait()
        pltpu.make_async_copy(ops_.at[pl.ds(0, CH)], psbuf[b], semw[b]).wait()
        pltpu.make_async_copy(opd.at[pl.ds(0, CH)], pdbuf[b], semw[b]).wait()

    g_issue(0, 0)

    @pl.loop(0, NCH, step=2)
    def _pair(j0):
        for b in range(2):
            j = j0 + b
            g_drain(b)

            @pl.when(j >= 1)
            def _():
                w_drain(1 - b)

            @pl.when(j + 1 < NCH)
            def _():
                g_issue(j + 1, 1 - b)

            w_issue(j, b)

    w_drain(1)


def _edge_body(as_ref, bd_ref, ps_ref, pd_ref, w1c_ref, w2_ref, b2_ref,
               wc_ref, bc_ref, m2_ref, rw_ref):
    i = pl.program_id(0)
    rel = ps_ref[...] - pd_ref[...]
    d2 = jnp.sum(rel * rel, axis=1, keepdims=True)
    x = (as_ref[...].astype(_f32) + bd_ref[...].astype(_f32)
         + d2 * w1c_ref[...])
    m1 = _silu(x)
    m2 = _silu(jnp.dot(m1, w2_ref[...], preferred_element_type=_f32)
               + b2_ref[...])
    w = jnp.dot(m2, wc_ref[...], preferred_element_type=_f32) + bc_ref[...]
    relw = rel * w
    relw = jnp.where(lax.broadcasted_iota(jnp.int32, (BE, 16), 1) == 3,
                     1.0, relw)
    rid = i * BE + lax.broadcasted_iota(jnp.int32, (BE, 1), 0)
    valid = rid < E
    m2_ref[...] = jnp.where(valid, m2, 0.0).astype(m2_ref.dtype)
    rw_ref[...] = jnp.where(valid, relw, 0.0)


def _k_edge(a_s, b_d, p_s, p_d, w1c, w2, b2, wc, bc):
    return pl.pallas_call(
        _edge_body,
        grid=(EP // BE,),
        in_specs=[
            pl.BlockSpec((BE, H), lambda i: (i, 0)),
            pl.BlockSpec((BE, H), lambda i: (i, 0)),
            pl.BlockSpec((BE, 16), lambda i: (i, 0)),
            pl.BlockSpec((BE, 16), lambda i: (i, 0)),
            pl.BlockSpec((1, H), lambda i: (0, 0)),
            pl.BlockSpec((H, H), lambda i: (0, 0)),
            pl.BlockSpec((1, H), lambda i: (0, 0)),
            pl.BlockSpec((H, 1), lambda i: (0, 0)),
            pl.BlockSpec((1, 1), lambda i: (0, 0)),
        ],
        out_specs=[
            pl.BlockSpec((BE, H), lambda i: (i, 0)),
            pl.BlockSpec((BE, 16), lambda i: (i, 0)),
        ],
        out_shape=[
            jax.ShapeDtypeStruct((EP, H), _f32),
            jax.ShapeDtypeStruct((EP, 16), _f32),
        ],
    )(a_s, b_d, p_s, p_d, w1c, w2, b2, wc, bc)


SCH = 128
NSCH = PW // SCH


@functools.partial(
    pl.kernel,
    out_type=[
        jax.ShapeDtypeStruct((2, N, H), _f32),
        jax.ShapeDtypeStruct((2, N, 16), _f32),
    ],
    mesh=_SC_MESH,
    scratch_types=[
        pltpu.VMEM((2, 128), jnp.int32),
        [pltpu.VMEM((SCH, H), _f32)] * 2,
        [pltpu.VMEM((SCH, 16), _f32)] * 2,
        pltpu.VMEM_SHARED((N, H), _f32),
        pltpu.VMEM_SHARED((N, 16), _f32),
        [pltpu.SemaphoreType.DMA] * 2,
    ],
    compiler_params=pltpu.CompilerParams(use_tc_tiling_on_sc=False),
)
def _scatter_sc(m2_hbm, rw_hbm, dst2_hbm,
                om, oc,
                ibuf, m2buf, rwbuf, accm, accc, seml):
    c = lax.axis_index("c")
    s = lax.axis_index("s")
    wid = s * 2 + c
    base = wid * PW
    rbase = wid * NSCH
    rows0 = s * RW

    def l_issue(j, b):
        off = base + j * SCH
        pltpu.async_copy(dst2_hbm.at[rbase + j], ibuf.at[b], seml[b])
        pltpu.async_copy(m2_hbm.at[pl.ds(off, SCH)], m2buf[b], seml[b])
        pltpu.async_copy(rw_hbm.at[pl.ds(off, SCH)], rwbuf[b], seml[b])

    def l_drain(b):
        pltpu.make_async_copy(dst2_hbm.at[0], ibuf.at[b], seml[b]).wait()
        pltpu.make_async_copy(m2_hbm.at[pl.ds(0, SCH)], m2buf[b],
                              seml[b]).wait()
        pltpu.make_async_copy(rw_hbm.at[pl.ds(0, SCH)], rwbuf[b],
                              seml[b]).wait()

    l_issue(0, 0)

    @pl.loop(0, SCH)
    def _zrow(i):
        for g in range(H // 16):
            m2buf[1][i, pl.ds(g * 16, 16)] = jnp.zeros((16,), _f32)
        rwbuf[1][i, pl.ds(0, 16)] = jnp.zeros((16,), _f32)

    for off, nrow in ((0, SCH), (SCH, SCH), (2 * SCH, SCH), (3 * SCH, SCH),
                      (4 * SCH, RW - 4 * SCH)):
        pltpu.sync_copy(m2buf[1].at[pl.ds(0, nrow)],
                        accm.at[pl.ds(rows0 + off, nrow)])
        pltpu.sync_copy(rwbuf[1].at[pl.ds(0, nrow)],
                        accc.at[pl.ds(rows0 + off, nrow)])
    plsc.subcore_barrier()

    @pl.loop(0, NSCH, step=2)
    def _pair(j0):
        for b in range(2):
            j = j0 + b
            l_drain(b)

            @pl.when(j + 1 < NSCH)
            def _():
                l_issue(j + 1, 1 - b)

            di = ibuf.at[b]
            pltpu.sync_copy(m2buf[b], accm.at[di], add=True)
            pltpu.sync_copy(rwbuf[b], accc.at[di], add=True)

    plsc.subcore_barrier()
    pltpu.sync_copy(accm.at[pl.ds(rows0, RW)], om.at[c, pl.ds(rows0, RW)])
    pltpu.sync_copy(accc.at[pl.ds(rows0, RW)], oc.at[c, pl.ds(rows0, RW)])


def _node_body(h_ref, aggp_ref, coordp_ref, pos_ref,
               wn1a_ref, wn1b_ref, bn1_ref, wn2_ref, bn2_ref,
               w1a_ref, w1b_ref, b1_ref,
               h_out, pos_out, a_out, b_out, *, last):
    agg = aggp_ref[0].astype(_f32) + aggp_ref[1].astype(_f32)
    hid = _silu(jnp.dot(h_ref[...], wn1a_ref[...], preferred_element_type=_f32)
                + jnp.dot(agg, wn1b_ref[...], preferred_element_type=_f32)
                + bn1_ref[...])
    hn = h_ref[...] + jnp.dot(hid, wn2_ref[...],
                              preferred_element_type=_f32) + bn2_ref[...]
    h_out[...] = hn
    if not last:
        coord = coordp_ref[0] + coordp_ref[1]
        deg = coord[:, 3:4] + 1.0
        posn = pos_ref[...] + coord / deg
        posn = jnp.where(
            lax.broadcasted_iota(jnp.int32, (BN, 16), 1) >= 3, 0.0, posn)
        pos_out[...] = posn
        a_out[...] = jnp.dot(hn, w1a_ref[...],
                             preferred_element_type=_f32).astype(a_out.dtype)
        b_out[...] = (jnp.dot(hn, w1b_ref[...], preferred_element_type=_f32)
                      + b1_ref[...]).astype(b_out.dtype)


def _k_node(h, aggp, coordp, pos, wn1a, wn1b, bn1, wn2, bn2, w1a, w1b, b1,
            last):
    n_out = 1 if last else 4
    out_specs = [pl.BlockSpec((BN, H), lambda i: (i, 0)),
                 pl.BlockSpec((BN, 16), lambda i: (i, 0)),
                 pl.BlockSpec((BN, H), lambda i: (i, 0)),
                 pl.BlockSpec((BN, H), lambda i: (i, 0))][:n_out]
    out_shape = [jax.ShapeDtypeStruct((N, H), _f32),
                 jax.ShapeDtypeStruct((N, 16), _f32),
                 jax.ShapeDtypeStruct((N, H), _f32),
                 jax.ShapeDtypeStruct((N, H), _f32)][:n_out]

    def body(*refs):
        ins = refs[:12]
        outs = list(refs[12:]) + [None] * (4 - n_out)
        _node_body(*ins, *outs, last=last)

    return pl.pallas_call(
        body,
        grid=(N // BN,),
        in_specs=[
            pl.BlockSpec((BN, H), lambda i: (i, 0)),
            pl.BlockSpec((2, BN, H), lambda i: (0, i, 0)),
            pl.BlockSpec((2, BN, 16), lambda i: (0, i, 0)),
            pl.BlockSpec((BN, 16), lambda i: (i, 0)),
            pl.BlockSpec((H, H), lambda i: (0, 0)),
            pl.BlockSpec((H, H), lambda i: (0, 0)),
            pl.BlockSpec((1, H), lambda i: (0, 0)),
            pl.BlockSpec((H, H), lambda i: (0, 0)),
            pl.BlockSpec((1, H), lambda i: (0, 0)),
            pl.BlockSpec((H, H), lambda i: (0, 0)),
            pl.BlockSpec((H, H), lambda i: (0, 0)),
            pl.BlockSpec((1, H), lambda i: (0, 0)),
        ],
        out_specs=out_specs,
        out_shape=out_shape,
    )(h, aggp, coordp, pos, wn1a, wn1b, bn1, wn2, bn2, w1a, w1b, b1)


def _pool_body(bt_ref, h_ref, o1_ref, bo1_ref, o2_ref, bo2_ref, out_ref,
               acc_ref):
    i = pl.program_id(0)

    @pl.when(i == 0)
    def _():
        acc_ref[...] = jnp.zeros((G, H), _f32)

    bt = bt_ref[...]
    oh = (lax.broadcasted_iota(jnp.int32, (BN, G), 1) == bt).astype(_f32)
    acc_ref[...] += lax.dot_general(oh, h_ref[...], (((0,), (0,)), ((), ())),
                                    preferred_element_type=_f32)

    @pl.when(i == pl.num_programs(0) - 1)
    def _():
        hg = acc_ref[...]
        out_ref[...] = jnp.dot(
            _silu(jnp.dot(hg, o1_ref[...], preferred_element_type=_f32)
                  + bo1_ref[...]),
            o2_ref[...], preferred_element_type=_f32) + bo2_ref[...]


def _k_pool(bt2d, h, o1, bo1, o2, bo2):
    return pl.pallas_call(
        _pool_body,
        grid=(N // BN,),
        in_specs=[
            pl.BlockSpec((BN, 1), lambda i: (i, 0)),
            pl.BlockSpec((BN, H), lambda i: (i, 0)),
            pl.BlockSpec((H, H), lambda i: (0, 0)),
            pl.BlockSpec((1, H), lambda i: (0, 0)),
            pl.BlockSpec((H, 1), lambda i: (0, 0)),
            pl.BlockSpec((1, 1), lambda i: (0, 0)),
        ],
        out_specs=pl.BlockSpec((G, 1), lambda i: (0, 0)),
        out_shape=jax.ShapeDtypeStruct((G, 1), _f32),
        scratch_shapes=[pltpu.VMEM((G, H), _f32)],
    )(bt2d, h, o1, bo1, o2, bo2)


def kernel(z, pos, edge_index, batch, params):
    conf_idx = 2
    pos3 = pos[:, conf_idx, :]
    posp = jnp.zeros((N, 16), _f32).at[:, :3].set(pos3)

    pad_idx = (jnp.arange(NPAD, dtype=jnp.int32) % N)
    srcp = jnp.concatenate([edge_index[0].astype(jnp.int32),
                            pad_idx]).reshape(EP // 128, 128)
    dstp = jnp.concatenate([edge_index[1].astype(jnp.int32),
                            pad_idx]).reshape(EP // 128, 128)

    z2d = z.astype(jnp.int32).reshape(N, 1)
    bt2d = batch.astype(jnp.int32).reshape(N, 1)
    z128 = jnp.zeros((N, H), _f32)
    z16 = jnp.zeros((N, 16), _f32)

    layers = params["layers"]

    def edge1_parts(layer):
        w1 = layer["edge1"]["W"]
        return (w1[:H], w1[H:2 * H], layer["edge1"]["b"].reshape(1, H),
                w1[2 * H].reshape(1, H))

    w1a0, w1b0, b10, _ = edge1_parts(layers[0])
    h, a, b = _k_init(z2d, params["embed"], w1a0, w1b0, b10)

    for li, layer in enumerate(layers):
        _, _, _, w1c = edge1_parts(layer)
        a_s, b_d, p_s, p_d = _gather_sc(a, b, posp, srcp, dstp)
        m2, relw = _k_edge(
            a_s, b_d, p_s, p_d, w1c,
            layer["edge2"]["W"], layer["edge2"]["b"].reshape(1, H),
            layer["coord"]["W"], layer["coord"]["b"].reshape(1, 1))
        aggp, coordp = _scatter_sc(m2, relw, dstp)

        last = li == len(layers) - 1
        wn1 = layer["node1"]["W"]
        if last:
            nw1a = nw1b = wn1[:H]
            nb1 = layer["node1"]["b"].reshape(1, H)
            outs = _k_node(h, aggp, coordp, posp,
                           wn1[:H], wn1[H:], nb1,
                           layer["node2"]["W"],
                           layer["node2"]["b"].reshape(1, H),
                           nw1a, nw1b, nb1, last=True)
            h = outs[0]
        else:
            w1an, w1bn, b1n, _ = edge1_parts(layers[li + 1])
            h, posp, a, b = _k_node(h, aggp, coordp, posp,
                                    wn1[:H], wn1[H:],
                                    layer["node1"]["b"].reshape(1, H),
                                    layer["node2"]["W"],
                                    layer["node2"]["b"].reshape(1, H),
                                    w1an, w1bn, b1n, last=False)

    out = _k_pool(bt2d, h, params["out1"]["W"],
                  params["out1"]["b"].reshape(1, H),
                  params["out2"]["W"], params["out2"]["b"].reshape(1, 1))
    return out
